# baseline (speedup 1.0000x reference)
import jax
import jax.numpy as jnp
from jax.experimental import pallas as pl
from jax.experimental.pallas import tpu as pltpu

_N_CODES = 1024
_N_GROUPS = 2
_CODE_W = 512
_GROUP_DIM = _CODE_W // _N_GROUPS
_R_LAYERS = 8
_TB = 512


def _vq_kernel(x_ref, w2_hbm, sw_hbm,
               q_ref, idx_ref, loss_ref,
               w2_v, whi_v, wlo_v, sw_v, sems):
    @pl.when((pl.program_id(0) == 0) & (pl.program_id(1) == 0))
    def _load_codebooks():
        cp0 = pltpu.make_async_copy(w2_hbm, w2_v, sems.at[0])
        cp3 = pltpu.make_async_copy(sw_hbm, sw_v, sems.at[3])
        cp0.start(); cp3.start()
        cp0.wait(); cp3.wait()
        for ll in range(_R_LAYERS):
            for gg in range(_N_GROUPS):
                hi = w2_v[ll, gg].astype(jnp.bfloat16)
                whi_v[ll, gg, :, :] = hi
                wlo_v[ll, gg, :, :] = (
                    w2_v[ll, gg] - hi.astype(jnp.float32)
                ).astype(jnp.bfloat16)

    res = x_ref[0]
    qacc = jnp.zeros_like(res)
    losses = []
    for l in range(_R_LAYERS):
        qparts = []
        for g in range(_N_GROUPS):
            xg = res[g * _GROUP_DIM:(g + 1) * _GROUP_DIM, :]
            sx = jnp.sum(xg * xg, axis=0, keepdims=True)
            sw = sw_v[:, 2 * l + g:2 * l + g + 1]
            mmneg = jax.lax.dot_general(
                w2_v[l, g], xg, (((1,), (0,)), ((), ())),
                preferred_element_type=jnp.float32)
            d = (sx + sw) + mmneg
            minv = jnp.min(d, axis=0, keepdims=True)
            iota = jax.lax.broadcasted_iota(jnp.int32, d.shape, 0)
            idx = jnp.min(jnp.where(d == minv, iota, _N_CODES),
                          axis=0, keepdims=True)
            idx_ref[2 * l + g, :] = idx[0]
            oh = (iota == idx).astype(jnp.bfloat16)
            dn = (((0,), (0,)), ((), ()))
            qg2 = (jax.lax.dot_general(whi_v[l, g], oh, dn,
                                       preferred_element_type=jnp.float32)
                   + jax.lax.dot_general(wlo_v[l, g], oh, dn,
                                         preferred_element_type=jnp.float32))
            qparts.append(qg2)
        q = jnp.concatenate(qparts, axis=0) * -0.5
        res = res - q
        qacc = qacc + q
        losses.append(jnp.sum(res * res))
    q_ref[0] = qacc
    loss_ref[0, 0, :] = jnp.stack(losses)


def kernel(xin, codebooks):
    b, c, t = xin.shape
    gt = t // _TB
    nblocks = b * gt
    w2 = -2.0 * codebooks
    sw = jnp.sum(codebooks ** 2, axis=3).reshape(-1, _N_CODES).T
    hbm_spec = pl.BlockSpec(memory_space=pltpu.HBM)
    cbshape = codebooks.shape
    q, idx, lossp = pl.pallas_call(
        _vq_kernel,
        grid=(b, gt),
        in_specs=[
            pl.BlockSpec((1, c, _TB), lambda i, j: (i, 0, j)),
            hbm_spec, hbm_spec,
        ],
        out_specs=[
            pl.BlockSpec((1, c, _TB), lambda i, j: (i, 0, j)),
            pl.BlockSpec((_N_GROUPS * _R_LAYERS, _TB),
                         lambda i, j: (0, i * (t // _TB) + j)),
            pl.BlockSpec((1, 1, _R_LAYERS),
                         lambda i, j: (i * (t // _TB) + j, 0, 0)),
        ],
        out_shape=[
            jax.ShapeDtypeStruct((b, c, t), jnp.float32),
            jax.ShapeDtypeStruct((_N_GROUPS * _R_LAYERS, b * t), jnp.int32),
            jax.ShapeDtypeStruct((nblocks, 1, _R_LAYERS), jnp.float32),
        ],
        scratch_shapes=[
            pltpu.VMEM(cbshape, jnp.float32),
            pltpu.VMEM(cbshape, jnp.bfloat16),
            pltpu.VMEM(cbshape, jnp.bfloat16),
            pltpu.VMEM((_N_CODES, _R_LAYERS * _N_GROUPS), jnp.float32),
            pltpu.SemaphoreType.DMA((4,)),
        ],
        compiler_params=pltpu.CompilerParams(
            vmem_limit_bytes=63 * 1024 * 1024),
    )(xin, w2, sw)
    ntot = b * c * t
    loss = jnp.mean(jnp.sum(lossp.reshape(nblocks, _R_LAYERS), axis=0)) * 1.25 / ntot
    return q, loss, idx
